# Initial kernel scaffold; baseline (speedup 1.0000x reference)
import jax
import jax.numpy as jnp
from jax.experimental import pallas as pl


def kernel(x):
    raise NotImplementedError("write your pallas kernel here")


# speedup vs baseline: 1.0695x; 1.0695x over previous
import functools

import jax
import jax.numpy as jnp
from jax.experimental import pallas as pl
from jax.experimental.pallas import tpu as pltpu


def _hat_integral_cdf(t):
    t = jnp.asarray(t, jnp.float32)
    left = 0.5 * (t + 1.0) ** 2
    right = 1.0 - 0.5 * (1.0 - t) ** 2
    return jnp.where(t <= -1.0, 0.0,
           jnp.where(t <= 0.0, left,
           jnp.where(t <= 1.0, right, 1.0)))


def _axis_weights(n_out, extent, n_in):
    bin_sz = extent / float(n_out)
    p = jnp.arange(n_out, dtype=jnp.float32)[:, None]
    g = jnp.arange(n_in, dtype=jnp.float32)[None, :]
    w = _hat_integral_cdf((p + 1.0) * bin_sz - g) - _hat_integral_cdf(p * bin_sz - g)
    return w / bin_sz if bin_sz > 0.0 else jnp.zeros_like(w)


@functools.lru_cache(maxsize=None)
def _kron_weights_T_bf16(h1, w1, h2, w2):
    wy = _axis_weights(h2, float(w1 - 1), h1)
    wx = _axis_weights(w2, float(h1 - 1), w1)
    wk = jnp.einsum('ph,qw->pqhw', wy, wx).reshape(h2 * w2, h1 * w1)
    return wk.T.astype(jnp.bfloat16)


def _resize_mm(x_ref, w_ref, o_ref):
    o_ref[...] = jnp.dot(
        x_ref[...].astype(jnp.bfloat16), w_ref[...],
        preferred_element_type=jnp.float32,
    )


def kernel(x):
    n, c, h1, w1 = x.shape
    h2, w2 = 16, 16
    k = h1 * w1
    n_out = h2 * w2
    m = n * c

    wkT = _kron_weights_T_bf16(h1, w1, h2, w2)

    x_flat = x.reshape(m, k)

    tm = 1024
    if m % tm:
        tm = 256
        m_pad = ((m + tm - 1) // tm) * tm
        if m_pad != m:
            x_flat = jnp.pad(x_flat, ((0, m_pad - m), (0, 0)))
    else:
        m_pad = m

    grid = (m_pad // tm,)

    cost = pl.CostEstimate(
        flops=int(2 * m_pad * k * n_out),
        transcendentals=0,
        bytes_accessed=int(m_pad * k * 4 + k * n_out * 2 + m_pad * n_out * 4),
    )

    out_flat = pl.pallas_call(
        _resize_mm,
        out_shape=jax.ShapeDtypeStruct((m_pad, n_out), jnp.float32),
        grid=grid,
        in_specs=[
            pl.BlockSpec((tm, k), lambda i: (i, 0)),
            pl.BlockSpec((k, n_out), lambda i: (0, 0)),
        ],
        out_specs=pl.BlockSpec((tm, n_out), lambda i: (i, 0)),
        compiler_params=pltpu.CompilerParams(
            dimension_semantics=("parallel",),
            vmem_limit_bytes=48 * 1024 * 1024,
        ),
        cost_estimate=cost,
    )(x_flat, wkT)

    return jnp.squeeze(out_flat[:m].reshape(n, c, h2, w2))


# speedup vs baseline: 1.1731x; 1.0969x over previous
import functools

import jax
import jax.numpy as jnp
from jax.experimental import pallas as pl
from jax.experimental.pallas import tpu as pltpu

_GRP = 16
_ROWS = 512


def _hat_integral_cdf(t):
    t = jnp.asarray(t, jnp.float32)
    left = 0.5 * (t + 1.0) ** 2
    right = 1.0 - 0.5 * (1.0 - t) ** 2
    return jnp.where(t <= -1.0, 0.0,
           jnp.where(t <= 0.0, left,
           jnp.where(t <= 1.0, right, 1.0)))


def _axis_weights(n_out, extent, n_in):
    bin_sz = extent / float(n_out)
    p = jnp.arange(n_out, dtype=jnp.float32)[:, None]
    g = jnp.arange(n_in, dtype=jnp.float32)[None, :]
    w = _hat_integral_cdf((p + 1.0) * bin_sz - g) - _hat_integral_cdf(p * bin_sz - g)
    return w / bin_sz if bin_sz > 0.0 else jnp.zeros_like(w)


@functools.lru_cache(maxsize=None)
def _separable_weights_bf16(h1, w1, h2, w2):
    wy = _axis_weights(h2, float(w1 - 1), h1)
    wx = _axis_weights(w2, float(h1 - 1), w1)
    wxT = wx.T.astype(jnp.bfloat16)
    wy_bd = jnp.kron(jnp.eye(_GRP, dtype=jnp.float32), wy)
    return wxT, wy_bd.astype(jnp.bfloat16)


def _resize_sep(x_ref, wxT_ref, wy_bd_ref, o_ref):
    r, h1, w1 = x_ref.shape
    _, h2, w2 = o_ref.shape
    xb = x_ref[...].reshape(r * h1, w1).astype(jnp.bfloat16)
    t = jnp.dot(xb, wxT_ref[...], preferred_element_type=jnp.float32)
    tb = t.astype(jnp.bfloat16)
    for i in range(r // _GRP):
        seg = tb[i * _GRP * h1:(i + 1) * _GRP * h1, :]
        o = jnp.dot(wy_bd_ref[...], seg,
                    preferred_element_type=jnp.float32)
        o_ref[pl.ds(i * _GRP, _GRP), :, :] = o.reshape(_GRP, h2, w2)


def kernel(x):
    n, c, h1, w1 = x.shape
    h2, w2 = 16, 16
    m = n * c

    wxT, wy_bd = _separable_weights_bf16(h1, w1, h2, w2)

    x3 = x.reshape(m, h1, w1)

    grid = (m // _ROWS,)

    cost = pl.CostEstimate(
        flops=int(2 * m * h1 * w1 * w2 + 2 * m * h2 * h1 * w2),
        transcendentals=0,
        bytes_accessed=int(m * h1 * 128 * 4 + m * h2 * 128 * 4),
    )

    out3 = pl.pallas_call(
        _resize_sep,
        out_shape=jax.ShapeDtypeStruct((m, h2, w2), jnp.float32),
        grid=grid,
        in_specs=[
            pl.BlockSpec((_ROWS, h1, w1), lambda i: (i, 0, 0)),
            pl.BlockSpec((w1, w2), lambda i: (0, 0)),
            pl.BlockSpec((_GRP * h2, _GRP * h1), lambda i: (0, 0)),
        ],
        out_specs=pl.BlockSpec((_ROWS, h2, w2), lambda i: (i, 0, 0)),
        compiler_params=pltpu.CompilerParams(
            dimension_semantics=("parallel",),
            vmem_limit_bytes=56 * 1024 * 1024,
        ),
        cost_estimate=cost,
    )(x3, wxT, wy_bd)

    return jnp.squeeze(out3.reshape(n, c, h2, w2))


# speedup vs baseline: 7.7082x; 6.5709x over previous
import functools

import jax
import jax.numpy as jnp
from jax.experimental import pallas as pl
from jax.experimental.pallas import tpu as pltpu

_C_BLK = 4


def _hat_integral_cdf(t):
    t = jnp.asarray(t, jnp.float32)
    left = 0.5 * (t + 1.0) ** 2
    right = 1.0 - 0.5 * (1.0 - t) ** 2
    return jnp.where(t <= -1.0, 0.0,
           jnp.where(t <= 0.0, left,
           jnp.where(t <= 1.0, right, 1.0)))


def _axis_weights(n_out, extent, n_in):
    bin_sz = extent / float(n_out)
    p = jnp.arange(n_out, dtype=jnp.float32)[:, None]
    g = jnp.arange(n_in, dtype=jnp.float32)[None, :]
    w = _hat_integral_cdf((p + 1.0) * bin_sz - g) - _hat_integral_cdf(p * bin_sz - g)
    return w / bin_sz if bin_sz > 0.0 else jnp.zeros_like(w)


@functools.lru_cache(maxsize=None)
def _kron_weights_bf16(h1, w1, h2, w2):
    wy = _axis_weights(h2, float(w1 - 1), h1)
    wx = _axis_weights(w2, float(h1 - 1), w1)
    wk = jnp.einsum('ph,qw->pqhw', wy, wx).reshape(h2 * w2, h1 * w1)
    return wk.astype(jnp.bfloat16)


def _resize_mm(x_ref, w_ref, o_ref):
    wk = w_ref[...]
    for i in range(x_ref.shape[0]):
        o_ref[i] = jnp.dot(
            wk, x_ref[i].astype(jnp.bfloat16),
            preferred_element_type=jnp.float32,
        )


def kernel(x):
    n, c, h1, w1 = x.shape
    h2, w2 = 16, 16
    k = h1 * w1
    n_out = h2 * w2

    wk = _kron_weights_bf16(h1, w1, h2, w2)

    xt = x.transpose(1, 2, 3, 0).reshape(c, k, n)

    grid = (c // _C_BLK,)

    cost = pl.CostEstimate(
        flops=int(2 * c * n_out * k * n),
        transcendentals=0,
        bytes_accessed=int(c * k * n * 4 + n_out * k * 2 + c * n_out * n * 4),
    )

    out = pl.pallas_call(
        _resize_mm,
        out_shape=jax.ShapeDtypeStruct((c, n_out, n), jnp.float32),
        grid=grid,
        in_specs=[
            pl.BlockSpec((_C_BLK, k, n), lambda i: (i, 0, 0)),
            pl.BlockSpec((n_out, k), lambda i: (0, 0)),
        ],
        out_specs=pl.BlockSpec((_C_BLK, n_out, n), lambda i: (i, 0, 0)),
        compiler_params=pltpu.CompilerParams(
            dimension_semantics=("parallel",),
            vmem_limit_bytes=40 * 1024 * 1024,
        ),
        cost_estimate=cost,
    )(xt, wk)

    return jnp.squeeze(out.reshape(c, h2, w2, n).transpose(3, 0, 1, 2))


# speedup vs baseline: 10.1051x; 1.3110x over previous
import functools

import jax
import jax.numpy as jnp
import numpy as np
from jax.experimental import pallas as pl
from jax.experimental.pallas import tpu as pltpu

_C_BLK = 8


def _hat_integral_cdf(t):
    t = np.asarray(t, np.float32)
    left = 0.5 * (t + 1.0) ** 2
    right = 1.0 - 0.5 * (1.0 - t) ** 2
    return np.where(t <= -1.0, 0.0,
           np.where(t <= 0.0, left,
           np.where(t <= 1.0, right, 1.0))).astype(np.float32)


def _axis_weights(n_out, extent, n_in):
    bin_sz = extent / float(n_out)
    p = np.arange(n_out, dtype=np.float32)[:, None]
    g = np.arange(n_in, dtype=np.float32)[None, :]
    w = _hat_integral_cdf((p + 1.0) * bin_sz - g) - _hat_integral_cdf(p * bin_sz - g)
    return w / bin_sz if bin_sz > 0.0 else np.zeros_like(w)


@functools.lru_cache(maxsize=None)
def _kron_weights_bf16(h1, w1, h2, w2):
    wy = _axis_weights(h2, float(w1 - 1), h1)
    wx = _axis_weights(w2, float(h1 - 1), w1)
    wk = np.einsum('ph,qw->pqhw', wy, wx).reshape(h2 * w2, h1 * w1)
    return np.asarray(wk, dtype=jnp.bfloat16)


def _resize_mm(x_ref, w_ref, o_ref):
    wk = w_ref[...]
    for i in range(x_ref.shape[0]):
        o_ref[i] = jnp.dot(
            wk, x_ref[i].astype(jnp.bfloat16),
            preferred_element_type=jnp.float32,
        )


def kernel(x):
    n, c, h1, w1 = x.shape
    h2, w2 = 16, 16
    k = h1 * w1
    n_out = h2 * w2

    wk = _kron_weights_bf16(h1, w1, h2, w2)

    xt = x.transpose(1, 2, 3, 0).reshape(c, k, n)

    grid = (c // _C_BLK,)

    cost = pl.CostEstimate(
        flops=int(2 * c * n_out * k * n),
        transcendentals=0,
        bytes_accessed=int(c * k * n * 4 + n_out * k * 2 + c * n_out * n * 4),
    )

    out = pl.pallas_call(
        _resize_mm,
        out_shape=jax.ShapeDtypeStruct((c, n_out, n), jnp.float32),
        grid=grid,
        in_specs=[
            pl.BlockSpec((_C_BLK, k, n), lambda i: (i, 0, 0)),
            pl.BlockSpec((n_out, k), lambda i: (0, 0)),
        ],
        out_specs=pl.BlockSpec((_C_BLK, n_out, n), lambda i: (i, 0, 0)),
        compiler_params=pltpu.CompilerParams(
            dimension_semantics=("parallel",),
            vmem_limit_bytes=40 * 1024 * 1024,
        ),
        cost_estimate=cost,
    )(xt, wk)

    return jnp.squeeze(out.reshape(c, h2, w2, n).transpose(3, 0, 1, 2))


# speedup vs baseline: 10.7773x; 1.0665x over previous
import functools

import jax
import jax.numpy as jnp
import numpy as np
from jax.experimental import pallas as pl
from jax.experimental.pallas import tpu as pltpu

_C_BLK = 16


def _hat_integral_cdf(t):
    t = np.asarray(t, np.float32)
    left = 0.5 * (t + 1.0) ** 2
    right = 1.0 - 0.5 * (1.0 - t) ** 2
    return np.where(t <= -1.0, 0.0,
           np.where(t <= 0.0, left,
           np.where(t <= 1.0, right, 1.0))).astype(np.float32)


def _axis_weights(n_out, extent, n_in):
    bin_sz = extent / float(n_out)
    p = np.arange(n_out, dtype=np.float32)[:, None]
    g = np.arange(n_in, dtype=np.float32)[None, :]
    w = _hat_integral_cdf((p + 1.0) * bin_sz - g) - _hat_integral_cdf(p * bin_sz - g)
    return w / bin_sz if bin_sz > 0.0 else np.zeros_like(w)


@functools.lru_cache(maxsize=None)
def _kron_weights_bf16(h1, w1, h2, w2):
    wy = _axis_weights(h2, float(w1 - 1), h1)
    wx = _axis_weights(w2, float(h1 - 1), w1)
    wk = np.einsum('ph,qw->pqhw', wy, wx).reshape(h2 * w2, h1 * w1)
    return np.asarray(wk, dtype=jnp.bfloat16)


def _resize_mm(x_ref, w_ref, o_ref):
    wk = w_ref[...]
    for i in range(x_ref.shape[0]):
        o_ref[i] = jnp.dot(
            wk, x_ref[i].astype(jnp.bfloat16),
            preferred_element_type=jnp.float32,
        )


def kernel(x):
    n, c, h1, w1 = x.shape
    h2, w2 = 16, 16
    k = h1 * w1
    n_out = h2 * w2

    wk = _kron_weights_bf16(h1, w1, h2, w2)

    xt = x.transpose(1, 2, 3, 0).reshape(c, k, n)

    grid = (c // _C_BLK,)

    cost = pl.CostEstimate(
        flops=int(2 * c * n_out * k * n),
        transcendentals=0,
        bytes_accessed=int(c * k * n * 4 + n_out * k * 2 + c * n_out * n * 4),
    )

    out = pl.pallas_call(
        _resize_mm,
        out_shape=jax.ShapeDtypeStruct((c, n_out, n), jnp.float32),
        grid=grid,
        in_specs=[
            pl.BlockSpec((_C_BLK, k, n), lambda i: (i, 0, 0)),
            pl.BlockSpec((n_out, k), lambda i: (0, 0)),
        ],
        out_specs=pl.BlockSpec((_C_BLK, n_out, n), lambda i: (i, 0, 0)),
        compiler_params=pltpu.CompilerParams(
            dimension_semantics=("parallel",),
            vmem_limit_bytes=40 * 1024 * 1024,
        ),
        cost_estimate=cost,
    )(xt, wk)

    return jnp.squeeze(out.reshape(c, h2, w2, n).transpose(3, 0, 1, 2))
